# baseline (speedup 1.0000x reference)
import functools

import jax
import jax.numpy as jnp
from jax import lax
from jax.experimental import pallas as pl
from jax.experimental.pallas import tpu as pltpu
from jax.experimental.pallas import tpu_sc as plsc

N = 10000
E = 320000
D = 128

NC = 2
NS = 16
NW = NC * NS

CW = 100
EPT = E // NW
CPT = EPT // CW
SB = 5
CPS = CPT // SB
RPT = N // NS
ZW = 80


def _make_scatter():
    mesh = plsc.VectorSubcoreMesh(
        core_axis_name="c", subcore_axis_name="s", num_cores=NC, num_subcores=NS
    )

    @functools.partial(
        pl.kernel,
        out_type=jax.ShapeDtypeStruct((NC, N, D), jnp.float32),
        mesh=mesh,
        scratch_types=[
            pltpu.VMEM((CPS, CW), jnp.int32),
            pltpu.VMEM((CPS, CW), jnp.int32),
            pltpu.VMEM((CW, D), jnp.float32),
            pltpu.VMEM((CW, D), jnp.float32),
            pltpu.VMEM((CW, D), jnp.float32),
            pltpu.VMEM_SHARED((N, D), jnp.float32),
            pltpu.SemaphoreType.DMA,
            pltpu.SemaphoreType.DMA,
            pltpu.SemaphoreType.DMA,
            pltpu.SemaphoreType.DMA,
            pltpu.SemaphoreType.DMA,
            pltpu.SemaphoreType.DMA,
        ],
    )
    def scatter_k(h_hbm, src_hbm, dst_hbm, out_hbm, sidx, didx, rows0, rows1,
                  rows2, acc, g0, g1, g2, s0, s1, s2):
        rows = rows0
        bufs = (rows0, rows1, rows2)
        gsems = (g0, g1, g2)
        ssems = (s0, s1, s2)
        cid = lax.axis_index("c")
        sid = lax.axis_index("s")
        wid = cid * NS + sid

        def zbody(k, carry):
            rows[k // 8, pl.ds((k % 8) * 16, 16)] = jnp.zeros((16,), jnp.float32)
            return carry

        lax.fori_loop(0, ZW * 8, zbody, 0)
        nblk = jnp.where(sid < NS - 1, 8, 5)
        base = sid * 8 * ZW

        def zsbody(t, carry):
            off = pl.multiple_of(base + t * ZW, ZW)
            pltpu.sync_copy(rows.at[pl.ds(0, ZW)], acc.at[pl.ds(off, ZW)])
            return carry

        lax.fori_loop(0, nblk, zsbody, 0)
        plsc.subcore_barrier()

        def gwait(b, sem):
            pltpu.make_async_copy(h_hbm.at[sidx.at[0]], b, sem).wait()

        def swait(b, sem):
            pltpu.make_async_copy(b, acc.at[pl.ds(0, CW)], sem).wait()

        def sblock(sb, carry):
            pltpu.sync_copy(src_hbm.at[wid, sb], sidx)
            pltpu.sync_copy(dst_hbm.at[wid, sb], didx)

            def block(c, gat):
                b = c % 3
                if c >= 1:
                    swait(bufs[(c - 1) % 3], ssems[(c - 1) % 3])
                if gat is not None:
                    gb = gat % 3
                    pltpu.async_copy(h_hbm.at[sidx.at[gat]], bufs[gb], gsems[gb])
                gwait(bufs[b], gsems[b])
                pltpu.async_copy(bufs[b], acc.at[didx.at[c]], ssems[b], add=True)

            pltpu.async_copy(h_hbm.at[sidx.at[0]], bufs[0], gsems[0])
            pltpu.async_copy(h_hbm.at[sidx.at[1]], bufs[1], gsems[1])
            block(0, 2)

            def tbody(t, carry2):
                c = 3 * t
                b0 = ssems[2]
                swait(bufs[2], b0)
                pltpu.async_copy(h_hbm.at[sidx.at[c + 2]], bufs[2], gsems[2])
                gwait(bufs[0], gsems[0])
                pltpu.async_copy(bufs[0], acc.at[didx.at[c]], ssems[0], add=True)

                swait(bufs[0], ssems[0])
                pltpu.async_copy(h_hbm.at[sidx.at[c + 3]], bufs[0], gsems[0])
                gwait(bufs[1], gsems[1])
                pltpu.async_copy(bufs[1], acc.at[didx.at[c + 1]], ssems[1], add=True)

                swait(bufs[1], ssems[1])
                pltpu.async_copy(h_hbm.at[sidx.at[c + 4]], bufs[1], gsems[1])
                gwait(bufs[2], gsems[2])
                pltpu.async_copy(bufs[2], acc.at[didx.at[c + 2]], ssems[2], add=True)
                return carry2

            block(1, 3)
            block(2, 4)
            lax.fori_loop(1, (CPS - 2) // 3, tbody, 0)
            block(CPS - 2, None)
            block(CPS - 1, None)
            swait(bufs[(CPS - 1) % 3], ssems[(CPS - 1) % 3])
            return carry

        lax.fori_loop(0, SB, sblock, 0)
        plsc.subcore_barrier()

        def wbody(t, carry):
            off = pl.multiple_of(base + t * ZW, ZW)
            pltpu.sync_copy(acc.at[pl.ds(off, ZW)], out_hbm.at[cid, pl.ds(off, ZW)])
            return carry

        lax.fori_loop(0, nblk, wbody, 0)

    return scatter_k


_scatter_cache = []


def _scatter(h, src3, dst3):
    if not _scatter_cache:
        _scatter_cache.append(_make_scatter())
    return _scatter_cache[0](h, src3, dst3)


def _matT(a, w):
    return lax.dot_general(
        a, w, (((1,), (1,)), ((), ())), preferred_element_type=jnp.float32
    )


def _bn_relu(y, g, b):
    m = jnp.mean(y, axis=0, keepdims=True)
    v = jnp.mean((y - m) ** 2, axis=0, keepdims=True)
    return jnp.maximum(g * (y - m) / jnp.sqrt(v + 1e-5) + b, 0.0)


def _proj0_body(x_ref, p0_ref, pb_ref, o_ref):
    o_ref[...] = _matT(x_ref[...], p0_ref[...]) + pb_ref[...]


def _proj1_body(h1_ref, p1_ref, sp0_ref, o_ref):
    o_ref[...] = sp0_ref[...] + _matT(h1_ref[...], p1_ref[...])


RB = 1000
NG = N // RB


def _accum_stats(y, s_ref, q_ref):
    @pl.when(pl.program_id(0) == 0)
    def _():
        s_ref[...] = jnp.zeros_like(s_ref)
        q_ref[...] = jnp.zeros_like(q_ref)

    s_ref[...] += jnp.sum(y, axis=0, keepdims=True)
    q_ref[...] += jnp.sum(y * y, axis=0, keepdims=True)


def _norm(y, s, q, g, b):
    m = s * (1.0 / N)
    v = q * (1.0 / N) - m * m
    return jnp.maximum(g * (y - m) * lax.rsqrt(v + 1e-5) + b, 0.0)


def _lin1_body(ei, eps_ref, acc_ref, x_ref, w1_ref, b1_ref, y_ref, s_ref, q_ref):
    pooled = acc_ref[0] + acc_ref[1] + (1.0 + eps_ref[ei]) * x_ref[...]
    y = _matT(pooled, w1_ref[...]) + b1_ref[...]
    y_ref[...] = y
    _accum_stats(y, s_ref, q_ref)


def _lin2_body(y_ref, s_ref, q_ref, g1_ref, be1_ref, w2_ref, b2_ref,
               y2_ref, s2_ref, q2_ref):
    h = _norm(y_ref[...], s_ref[...], q_ref[...], g1_ref[...], be1_ref[...])
    y2 = _matT(h, w2_ref[...]) + b2_ref[...]
    y2_ref[...] = y2
    _accum_stats(y2, s2_ref, q2_ref)


def _out0_body(y2_ref, s_ref, q_ref, bg_ref, bb_ref, h_ref):
    h_ref[...] = _norm(y2_ref[...], s_ref[...], q_ref[...], bg_ref[...],
                       bb_ref[...])


def _out1_body(y2_ref, s_ref, q_ref, bg_ref, bb_ref, sp_ref, p2_ref, o_ref):
    h2 = _norm(y2_ref[...], s_ref[...], q_ref[...], bg_ref[...], bb_ref[...])
    o_ref[...] = sp_ref[...] + _matT(h2, p2_ref[...])


def _tc_call(body, n_in, out_shapes, smem_first=True):
    smem = pl.BlockSpec(memory_space=pltpu.SMEM)
    head = [smem] if smem_first else [pl.BlockSpec()]
    return pl.pallas_call(
        body,
        in_specs=head + [pl.BlockSpec()] * (n_in - 1),
        out_specs=[pl.BlockSpec()] * len(out_shapes),
        out_shape=[jax.ShapeDtypeStruct(s, jnp.float32) for s in out_shapes],
        compiler_params=pltpu.CompilerParams(
            vmem_limit_bytes=120 * 1024 * 1024,
        ),
    )


_BLK = lambda: pl.BlockSpec((RB, D), lambda i: (i, 0))
_ACC = lambda: pl.BlockSpec((NC, RB, D), lambda i: (0, i, 0))
_ONE = lambda r=1: pl.BlockSpec((r, D), lambda i: (0, 0))
_W = lambda: pl.BlockSpec((D, D), lambda i: (0, 0))


def _grid_call(body, in_specs, n_out_blk, n_out_stat):
    out_specs = [_BLK() for _ in range(n_out_blk)] + [_ONE() for _ in range(n_out_stat)]
    out_shape = ([jax.ShapeDtypeStruct((N, D), jnp.float32)] * n_out_blk
                 + [jax.ShapeDtypeStruct((1, D), jnp.float32)] * n_out_stat)
    return pl.pallas_call(
        body,
        grid=(NG,),
        in_specs=in_specs,
        out_specs=out_specs,
        out_shape=out_shape,
        compiler_params=pltpu.CompilerParams(
            vmem_limit_bytes=120 * 1024 * 1024,
        ),
    )


def kernel(x, edge_index, eps, m0_W1, m0_b1, m0_g1, m0_be1, m0_W2, m0_b2,
           bn0_g, bn0_b, m1_W1, m1_b1, m1_g1, m1_be1, m1_W2, m1_b2,
           bn1_g, bn1_b, p0_W, p0_b, p1_W, p1_b, p2_W, p2_b):
    src3 = edge_index[0].reshape(NW, SB, CPS, CW)
    dst3 = edge_index[1].reshape(NW, SB, CPS, CW)

    r = lambda a: a.reshape(1, D)

    smem = pl.BlockSpec(memory_space=pltpu.SMEM)

    def mlp_front(ei, acc, hin, W1, b1, g1, be1, W2, b2):
        y, s, q = _grid_call(
            functools.partial(_lin1_body, ei),
            [smem, _ACC(), _BLK(), _W(), _ONE()], 1, 2,
        )(eps, acc, hin, W1, r(b1))
        y2, s2, q2 = _grid_call(
            _lin2_body,
            [_BLK(), _ONE(), _ONE(), _ONE(), _ONE(), _W(), _ONE()], 1, 2,
        )(y, s, q, r(g1), r(be1), W2, r(b2))
        return y2, s2, q2

    acc0 = _scatter(x, src3, dst3)
    (sp0,) = _tc_call(_proj0_body, 3, [(N, D)], smem_first=False)(
        x, p0_W, r(p0_b + p1_b + p2_b)
    )
    y2, s2, q2 = mlp_front(0, acc0, x, m0_W1, m0_b1, m0_g1, m0_be1, m0_W2, m0_b2)
    (h1,) = _grid_call(
        _out0_body, [_BLK(), _ONE(), _ONE(), _ONE(), _ONE()], 1, 0
    )(y2, s2, q2, r(bn0_g), r(bn0_b))

    acc1 = _scatter(h1, src3, dst3)
    (sp01,) = _tc_call(_proj1_body, 3, [(N, D)], smem_first=False)(h1, p1_W, sp0)
    y2b, s2b, q2b = mlp_front(1, acc1, h1, m1_W1, m1_b1, m1_g1, m1_be1,
                              m1_W2, m1_b2)
    (score,) = _grid_call(
        _out1_body, [_BLK(), _ONE(), _ONE(), _ONE(), _ONE(), _BLK(), _W()], 1, 0
    )(y2b, s2b, q2b, r(bn1_g), r(bn1_b), sp01, p2_W)
    return score


# speedup vs baseline: 1.1166x; 1.1166x over previous
import functools

import jax
import jax.numpy as jnp
from jax import lax
from jax.experimental import pallas as pl
from jax.experimental.pallas import tpu as pltpu
from jax.experimental.pallas import tpu_sc as plsc

N = 10000
E = 320000
D = 128

NC = 2
NS = 16
NW = NC * NS

CW = 100
EPT = E // NW
CPT = EPT // CW
SB = 5
CPS = CPT // SB
RPT = N // NS
ZW = 80


def _make_scatter():
    mesh = plsc.VectorSubcoreMesh(
        core_axis_name="c", subcore_axis_name="s", num_cores=NC, num_subcores=NS
    )

    @functools.partial(
        pl.kernel,
        out_type=jax.ShapeDtypeStruct((NC, N, D), jnp.float32),
        mesh=mesh,
        scratch_types=[
            pltpu.VMEM((CPS, CW), jnp.int32),
            pltpu.VMEM((CPS, CW), jnp.int32),
            pltpu.VMEM((CW, D), jnp.float32),
            pltpu.VMEM((CW, D), jnp.float32),
            pltpu.VMEM((CW, D), jnp.float32),
            pltpu.VMEM_SHARED((N, D), jnp.float32),
            pltpu.SemaphoreType.DMA,
            pltpu.SemaphoreType.DMA,
            pltpu.SemaphoreType.DMA,
            pltpu.SemaphoreType.DMA,
            pltpu.SemaphoreType.DMA,
            pltpu.SemaphoreType.DMA,
        ],
    )
    def scatter_k(h_hbm, src_hbm, dst_hbm, out_hbm, sidx, didx, rows0, rows1,
                  rows2, acc, g0, g1, g2, s0, s1, s2):
        rows = rows0
        bufs = (rows0, rows1, rows2)
        gsems = (g0, g1, g2)
        ssems = (s0, s1, s2)
        cid = lax.axis_index("c")
        sid = lax.axis_index("s")
        wid = cid * NS + sid

        def zbody(k, carry):
            rows[k // 8, pl.ds((k % 8) * 16, 16)] = jnp.zeros((16,), jnp.float32)
            return carry

        lax.fori_loop(0, ZW * 8, zbody, 0)
        nblk = jnp.where(sid < NS - 1, 8, 5)
        base = sid * 8 * ZW

        def zsbody(t, carry):
            off = pl.multiple_of(base + t * ZW, ZW)
            pltpu.sync_copy(rows.at[pl.ds(0, ZW)], acc.at[pl.ds(off, ZW)])
            return carry

        lax.fori_loop(0, nblk, zsbody, 0)
        plsc.subcore_barrier()

        def gwait(b, sem):
            pltpu.make_async_copy(h_hbm.at[sidx.at[0]], b, sem).wait()

        def swait(b, sem):
            pltpu.make_async_copy(b, acc.at[pl.ds(0, CW)], sem).wait()

        def sblock(sb, carry):
            pltpu.sync_copy(src_hbm.at[wid, sb], sidx)
            pltpu.sync_copy(dst_hbm.at[wid, sb], didx)

            def block(c, gat):
                b = c % 3
                if c >= 1:
                    swait(bufs[(c - 1) % 3], ssems[(c - 1) % 3])
                if gat is not None:
                    gb = gat % 3
                    pltpu.async_copy(h_hbm.at[sidx.at[gat]], bufs[gb], gsems[gb])
                gwait(bufs[b], gsems[b])
                pltpu.async_copy(bufs[b], acc.at[didx.at[c]], ssems[b], add=True)

            pltpu.async_copy(h_hbm.at[sidx.at[0]], bufs[0], gsems[0])
            pltpu.async_copy(h_hbm.at[sidx.at[1]], bufs[1], gsems[1])
            block(0, 2)

            def tbody(t, carry2):
                c = 3 * t
                b0 = ssems[2]
                swait(bufs[2], b0)
                pltpu.async_copy(h_hbm.at[sidx.at[c + 2]], bufs[2], gsems[2])
                gwait(bufs[0], gsems[0])
                pltpu.async_copy(bufs[0], acc.at[didx.at[c]], ssems[0], add=True)

                swait(bufs[0], ssems[0])
                pltpu.async_copy(h_hbm.at[sidx.at[c + 3]], bufs[0], gsems[0])
                gwait(bufs[1], gsems[1])
                pltpu.async_copy(bufs[1], acc.at[didx.at[c + 1]], ssems[1], add=True)

                swait(bufs[1], ssems[1])
                pltpu.async_copy(h_hbm.at[sidx.at[c + 4]], bufs[1], gsems[1])
                gwait(bufs[2], gsems[2])
                pltpu.async_copy(bufs[2], acc.at[didx.at[c + 2]], ssems[2], add=True)
                return carry2

            block(1, 3)
            block(2, 4)
            lax.fori_loop(1, (CPS - 2) // 3, tbody, 0)
            block(CPS - 2, None)
            block(CPS - 1, None)
            swait(bufs[(CPS - 1) % 3], ssems[(CPS - 1) % 3])
            return carry

        lax.fori_loop(0, SB, sblock, 0)
        plsc.subcore_barrier()

        def wbody(t, carry):
            off = pl.multiple_of(base + t * ZW, ZW)
            pltpu.sync_copy(acc.at[pl.ds(off, ZW)], out_hbm.at[cid, pl.ds(off, ZW)])
            return carry

        lax.fori_loop(0, nblk, wbody, 0)

    return scatter_k


_scatter_cache = []


def _scatter(h, src3, dst3):
    if not _scatter_cache:
        _scatter_cache.append(_make_scatter())
    return _scatter_cache[0](h, src3, dst3)


def _matT(a, w):
    return lax.dot_general(
        a, w, (((1,), (1,)), ((), ())), preferred_element_type=jnp.float32
    )


def _bn_relu(y, g, b):
    m = jnp.mean(y, axis=0, keepdims=True)
    v = jnp.mean((y - m) ** 2, axis=0, keepdims=True)
    return jnp.maximum(g * (y - m) / jnp.sqrt(v + 1e-5) + b, 0.0)


def _proj0_body(x_ref, p0_ref, pb_ref, o_ref):
    o_ref[...] = _matT(x_ref[...], p0_ref[...]) + pb_ref[...]


def _proj1_body(h1_ref, p1_ref, sp0_ref, o_ref):
    o_ref[...] = sp0_ref[...] + _matT(h1_ref[...], p1_ref[...])


def _mlp0_body(eps_ref, acc_ref, x_ref, w1_ref, b1_ref, g1_ref, be1_ref,
               w2_ref, b2_ref, bg_ref, bb_ref, h1_ref):
    pooled = acc_ref[0] + acc_ref[1] + (1.0 + eps_ref[0]) * x_ref[...]
    y = _matT(pooled, w1_ref[...]) + b1_ref[...]
    h = _bn_relu(y, g1_ref[...], be1_ref[...])
    y2 = _matT(h, w2_ref[...]) + b2_ref[...]
    h1_ref[...] = _bn_relu(y2, bg_ref[...], bb_ref[...])


def _mlp1_body(eps_ref, acc_ref, h1_ref, sp01_ref, w1_ref, b1_ref, g1_ref,
               be1_ref, w2_ref, b2_ref, bg_ref, bb_ref, p2_ref, score_ref):
    pooled = acc_ref[0] + acc_ref[1] + (1.0 + eps_ref[1]) * h1_ref[...]
    y = _matT(pooled, w1_ref[...]) + b1_ref[...]
    h = _bn_relu(y, g1_ref[...], be1_ref[...])
    y2 = _matT(h, w2_ref[...]) + b2_ref[...]
    h2 = _bn_relu(y2, bg_ref[...], bb_ref[...])
    score_ref[...] = sp01_ref[...] + _matT(h2, p2_ref[...])


def _tc_call(body, n_in, out_shapes, smem_first=True):
    smem = pl.BlockSpec(memory_space=pltpu.SMEM)
    head = [smem] if smem_first else [pl.BlockSpec()]
    return pl.pallas_call(
        body,
        in_specs=head + [pl.BlockSpec()] * (n_in - 1),
        out_specs=[pl.BlockSpec()] * len(out_shapes),
        out_shape=[jax.ShapeDtypeStruct(s, jnp.float32) for s in out_shapes],
        compiler_params=pltpu.CompilerParams(
            vmem_limit_bytes=120 * 1024 * 1024,
        ),
    )


def kernel(x, edge_index, eps, m0_W1, m0_b1, m0_g1, m0_be1, m0_W2, m0_b2,
           bn0_g, bn0_b, m1_W1, m1_b1, m1_g1, m1_be1, m1_W2, m1_b2,
           bn1_g, bn1_b, p0_W, p0_b, p1_W, p1_b, p2_W, p2_b):
    src3 = edge_index[0].reshape(NW, SB, CPS, CW)
    dst3 = edge_index[1].reshape(NW, SB, CPS, CW)

    r = lambda a: a.reshape(1, D)

    acc0 = _scatter(x, src3, dst3)
    (sp0,) = _tc_call(_proj0_body, 3, [(N, D)], smem_first=False)(
        x, p0_W, r(p0_b + p1_b + p2_b)
    )
    (h1,) = _tc_call(_mlp0_body, 11, [(N, D)])(
        eps, acc0, x, m0_W1, r(m0_b1), r(m0_g1), r(m0_be1), m0_W2, r(m0_b2),
        r(bn0_g), r(bn0_b)
    )
    acc1 = _scatter(h1, src3, dst3)
    (sp01,) = _tc_call(_proj1_body, 3, [(N, D)], smem_first=False)(h1, p1_W, sp0)
    (score,) = _tc_call(_mlp1_body, 13, [(N, D)])(
        eps, acc1, h1, sp01, m1_W1, r(m1_b1), r(m1_g1), r(m1_be1), m1_W2,
        r(m1_b2), r(bn1_g), r(bn1_b), p2_W
    )
    return score
